# baseline (speedup 1.0000x reference)
import functools

import jax
import jax.numpy as jnp
from jax import lax
from jax.experimental import pallas as pl
from jax.experimental.pallas import tpu as pltpu
from jax.experimental.pallas import tpu_sc as plsc

_BT = 4096
_H = 200
_D = 64
_TPAD = 137


def _build(BT, H, D):
    info = plsc.get_sparse_core_info()
    NC, NS, L = info.num_cores, info.num_subcores, info.num_lanes
    NW = NC * NS
    BBLK = BT // NW
    NBUF = 4
    NOUT = H // NBUF
    NCH = D // L

    mesh = plsc.VectorSubcoreMesh(core_axis_name="c", subcore_axis_name="s")

    @functools.partial(
        pl.kernel,
        out_type=jax.ShapeDtypeStruct((H, D // 8, NW, 8, BBLK), jnp.float32),
        mesh=mesh,
        scratch_types=[
            pltpu.VMEM((H, BBLK), jnp.int32),
            pltpu.VMEM((NBUF, BBLK, D), jnp.float32),
            pltpu.VMEM((NBUF, D // 8, 8, _TPAD), jnp.float32),
            pltpu.SemaphoreType.DMA((NBUF,)),
            pltpu.SemaphoreType.DMA((NBUF,)),
        ],
        compiler_params=pltpu.CompilerParams(
            use_tc_tiling_on_sc=False, needs_layout_passes=False
        ),
    )
    def gather_kernel(xt_hbm, table_hbm, out_hbm, idx_v, gbuf, tbuf, gsem, ssem):
        wid = lax.axis_index("s") * NC + lax.axis_index("c")
        b0 = wid * BBLK

        pltpu.sync_copy(xt_hbm.at[:, pl.ds(b0, BBLK)], idx_v)

        iota = lax.iota(jnp.int32, L)
        ivecs = [(iota + c * L) >> 3 for c in range(NCH)]
        dvecs = [(iota + c * L) & 7 for c in range(NCH)]

        def gather_h(h, s):
            pltpu.async_copy(table_hbm.at[idx_v.at[h]], gbuf.at[s], gsem.at[s])

        def wait_gather(s):
            pltpu.make_async_copy(
                table_hbm.at[idx_v.at[0]], gbuf.at[s], gsem.at[s]
            ).wait()

        def store_h(h, s):
            pltpu.async_copy(
                tbuf.at[s, :, :, pl.ds(0, BBLK)],
                out_hbm.at[h, :, wid],
                ssem.at[s],
            )

        def wait_store(s):
            pltpu.make_async_copy(
                tbuf.at[s, :, :, pl.ds(0, BBLK)],
                out_hbm.at[0, :, wid],
                ssem.at[s],
            ).wait()

        UNROLL = 8

        def transpose_block(s):
            def bloop(g, carry):
                base = g * UNROLL
                for u in range(UNROLL):
                    bp = base + u
                    col = jnp.full((L,), bp, jnp.int32)
                    for c in range(NCH):
                        vals = gbuf[s, bp, pl.ds(c * L, L)]
                        plsc.store_scatter(
                            tbuf.at[s], [ivecs[c], dvecs[c], col], vals
                        )
                return carry

            lax.fori_loop(0, BBLK // UNROLL, bloop, 0)

        for s in range(NBUF):
            gather_h(s, s)

        def outer(o, carry):
            for s in range(NBUF):
                h = o * NBUF + s
                wait_gather(s)

                @pl.when(h >= NBUF)
                def _():
                    wait_store(s)

                transpose_block(s)

                @pl.when(h + NBUF < H)
                def _():
                    gather_h(h + NBUF, s)

                store_h(h, s)
            return carry

        lax.fori_loop(0, NOUT, outer, 0)

        for s in range(NBUF):
            wait_store(s)

    return gather_kernel


_gather = _build(_BT, _H, _D)


@jax.jit
def kernel(x, table):
    out5 = _gather(x.T, table)
    t = jnp.transpose(out5, (0, 1, 3, 2, 4))
    t = t.reshape(_H, _D, _BT)
    return jnp.transpose(t, (2, 0, 1))


# speedup vs baseline: 2.7205x; 2.7205x over previous
import functools

import jax
import jax.numpy as jnp
from jax import lax
from jax.experimental import pallas as pl
from jax.experimental.pallas import tpu as pltpu
from jax.experimental.pallas import tpu_sc as plsc

_BT = 4096
_H = 200
_D = 64
_TPAD = 137


def _build(BT, H, D):
    info = plsc.get_sparse_core_info()
    NC, NS, L = info.num_cores, info.num_subcores, info.num_lanes
    NW = NC * NS
    BBLK = BT // NW
    NBUF = 4
    NOUT = H // NBUF
    NCH = D // L

    mesh = plsc.VectorSubcoreMesh(core_axis_name="c", subcore_axis_name="s")

    @functools.partial(
        pl.kernel,
        out_type=jax.ShapeDtypeStruct((H, D // 8, NW, 8, BBLK), jnp.float32),
        mesh=mesh,
        scratch_types=[
            pltpu.VMEM((H, BBLK), jnp.int32),
            pltpu.VMEM((NBUF, BBLK, D), jnp.float32),
            pltpu.VMEM((NBUF, D // 8, 8, _TPAD), jnp.float32),
            pltpu.SemaphoreType.DMA((NBUF,)),
            pltpu.SemaphoreType.DMA((NBUF,)),
        ],
        compiler_params=pltpu.CompilerParams(
            use_tc_tiling_on_sc=False, needs_layout_passes=False
        ),
    )
    def gather_kernel(xt_hbm, table_hbm, out_hbm, idx_v, gbuf, tbuf, gsem, ssem):
        wid = lax.axis_index("s") * NC + lax.axis_index("c")
        b0 = wid * BBLK

        pltpu.sync_copy(xt_hbm.at[:, pl.ds(b0, BBLK)], idx_v)

        iota = lax.iota(jnp.int32, L)
        ivecs = [(iota + c * L) >> 3 for c in range(NCH)]
        dvecs = [(iota + c * L) & 7 for c in range(NCH)]

        def gather_h(h, s):
            pltpu.async_copy(table_hbm.at[idx_v.at[h]], gbuf.at[s], gsem.at[s])

        def wait_gather(s):
            pltpu.make_async_copy(
                table_hbm.at[idx_v.at[0]], gbuf.at[s], gsem.at[s]
            ).wait()

        def store_h(h, s):
            pltpu.async_copy(
                tbuf.at[s, :, :, pl.ds(0, BBLK)],
                out_hbm.at[h, :, wid],
                ssem.at[s],
            )

        def wait_store(s):
            pltpu.make_async_copy(
                tbuf.at[s, :, :, pl.ds(0, BBLK)],
                out_hbm.at[0, :, wid],
                ssem.at[s],
            ).wait()

        def transpose_block(s):
            @plsc.parallel_loop(0, BBLK, unroll=8)
            def bloop(bp):
                col = jnp.full((L,), bp, jnp.int32)
                for c in range(NCH):
                    vals = gbuf[s, bp, pl.ds(c * L, L)]
                    plsc.store_scatter(
                        tbuf.at[s], [ivecs[c], dvecs[c], col], vals
                    )

        for s in range(NBUF):
            gather_h(s, s)

        def outer(o, carry):
            for s in range(NBUF):
                h = o * NBUF + s
                wait_gather(s)

                @pl.when(h >= NBUF)
                def _():
                    wait_store(s)

                transpose_block(s)

                @pl.when(h + NBUF < H)
                def _():
                    gather_h(h + NBUF, s)

                store_h(h, s)
            return carry

        lax.fori_loop(0, NOUT, outer, 0)

        for s in range(NBUF):
            wait_store(s)

    return gather_kernel


_gather = _build(_BT, _H, _D)


@jax.jit
def kernel(x, table):
    out5 = _gather(x.T, table)
    t = jnp.transpose(out5, (0, 1, 3, 2, 4))
    t = t.reshape(_H, _D, _BT)
    return jnp.transpose(t, (2, 0, 1))
